# baseline (speedup 1.0000x reference)
import functools

import jax
import jax.numpy as jnp
from jax import lax
from jax.experimental import pallas as pl
from jax.experimental.pallas import tpu as pltpu
from jax.experimental.pallas import tpu_sc as plsc

H = 128
EPS = 1e-5
NC = 2
NS = 16
NW = NC * NS
K_GATHER = 40
K_SCATTER = 80
SEG = 2

F32 = jnp.float32


def _ln_relu(h, g, b):
    mu = jnp.mean(h, axis=-1, keepdims=True)
    var = jnp.mean((h - mu) ** 2, axis=-1, keepdims=True)
    return jnp.maximum((h - mu) * lax.rsqrt(var + EPS) * g + b, 0.0)


def _node_pre_body(nf, wn, bn, gn, ben, w1, w2, bm, x_o, a_o, b_o):
    h = jnp.dot(nf[...], wn[...], preferred_element_type=F32) + bn[...]
    x = _ln_relu(h, gn[...], ben[...])
    x_o[...] = x
    a_o[...] = jnp.dot(x, w1[...], preferred_element_type=F32) + bm[...]
    b_o[...] = jnp.dot(x, w2[...], preferred_element_type=F32)


def _node_pre(nf, wn, bn, gn, ben, w1, w2, bm):
    n = nf.shape[0]
    blk = 2000
    grid = n // blk
    full = lambda i: (0, 0)
    chunk = lambda i: (i, 0)
    specs = [
        pl.BlockSpec((blk, H), chunk),
        pl.BlockSpec((H, H), full),
        pl.BlockSpec((1, H), full),
        pl.BlockSpec((1, H), full),
        pl.BlockSpec((1, H), full),
        pl.BlockSpec((H, H), full),
        pl.BlockSpec((H, H), full),
        pl.BlockSpec((1, H), full),
    ]
    out = jax.ShapeDtypeStruct((n, H), F32)
    return pl.pallas_call(
        _node_pre_body,
        grid=(grid,),
        in_specs=specs,
        out_specs=[pl.BlockSpec((blk, H), chunk)] * 3,
        out_shape=[out, out, out],
    )(nf, wn, bn, gn, ben, w1, w2, bm)


def _sc_gather_kernel(n_nodes, seg_off, seg_e):
    K_EDGE = K_GATHER
    epw = seg_e // NW
    chunks = epw // K_EDGE
    S = 4
    n_pad = ((n_nodes + NS * 8 - 1) // (NS * 8)) * (NS * 8)
    rpt = n_pad // NS
    mesh = plsc.VectorSubcoreMesh(core_axis_name="c", subcore_axis_name="s")
    out = jax.ShapeDtypeStruct((seg_e, H), F32)

    @functools.partial(
        pl.kernel,
        mesh=mesh,
        out_type=[out, out],
        scratch_types=(
            [pltpu.VMEM_SHARED((n_pad, H), F32)]
            + [pltpu.VMEM((epw,), jnp.int32) for _ in range(2)]
            + [pltpu.VMEM((K_EDGE, H), F32) for _ in range(S)]
            + [pltpu.SemaphoreType.DMA for _ in range(2 * S)]
        ),
    )
    def sc_gather(a_hbm, b_hbm, row_hbm, col_hbm, ar_hbm, bc_hbm, *scr):
        tab_sh = scr[0]
        idx_r_all, idx_c_all = scr[1:3]
        buf = scr[3:3 + S]
        sg = scr[3 + S:3 + 2 * S]
        sw = scr[3 + 2 * S:3 + 3 * S]
        ci = lax.axis_index("c")
        si = lax.axis_index("s")
        base_w = (ci * NS + si) * epw
        base_i = seg_off + base_w
        pltpu.sync_copy(row_hbm.at[pl.ds(base_i, epw)], idx_r_all)
        pltpu.sync_copy(col_hbm.at[pl.ds(base_i, epw)], idx_c_all)

        def one_pass(tab_hbm, idx_all, out_hbm):
            pltpu.sync_copy(tab_hbm.at[pl.ds(si * rpt, rpt)],
                            tab_sh.at[pl.ds(si * rpt, rpt)])
            plsc.subcore_barrier()

            def issue_g(i, s):
                pltpu.async_copy(
                    tab_sh.at[idx_all.at[pl.ds(i * K_EDGE, K_EDGE)]],
                    buf[s], sg[s])

            def wait_g(i, s):
                pltpu.make_async_copy(
                    tab_sh.at[idx_all.at[pl.ds(i * K_EDGE, K_EDGE)]],
                    buf[s], sg[s]).wait()

            def issue_w(i, s):
                base = base_w + i * K_EDGE
                pltpu.async_copy(buf[s], out_hbm.at[pl.ds(base, K_EDGE)],
                                 sw[s])

            def wait_w(i, s):
                base = base_w + i * K_EDGE
                pltpu.make_async_copy(
                    buf[s], out_hbm.at[pl.ds(base, K_EDGE)], sw[s]).wait()

            assert chunks >= 6
            issue_g(0, 0)
            issue_g(1, 1)
            issue_g(2, 2)
            wait_g(0, 0)
            issue_w(0, 0)
            issue_g(3, 3)
            wait_g(1, 1)
            issue_w(1, 1)
            wait_w(0, 0)
            issue_g(4, 0)

            def body(i, carry):
                for k in range(S):
                    j = 4 * i + 2 + k
                    s = (2 + k) % S
                    sp = (s + 3) % S
                    wait_g(j, s)
                    issue_w(j, s)
                    wait_w(j - 1, sp)
                    issue_g(j + 3, sp)
                return carry

            niter = (chunks - 5) // 4
            lax.fori_loop(0, niter, body, 0)
            for t in range((chunks - 5) % 4):
                j = 2 + 4 * niter + t
                s = j % S
                sp = (j + 3) % S
                wait_g(j, s)
                issue_w(j, s)
                wait_w(j - 1, sp)
                issue_g(j + 3, sp)
            for j in range(chunks - 3, chunks):
                wait_g(j, j % S)
                issue_w(j, j % S)
            for j in range(chunks - 4, chunks):
                wait_w(j, j % S)
            plsc.subcore_barrier()

        one_pass(a_hbm, idx_r_all, ar_hbm)
        one_pass(b_hbm, idx_c_all, bc_hbm)

    return sc_gather


def _msg_body(ef, ar, bc, we, be_, ge, bee, w2, gm, bem, msg_o):
    h = jnp.dot(ef[...], we[...], preferred_element_type=F32) + be_[...]
    ea = _ln_relu(h, ge[...], bee[...])
    v = ar[...] + bc[...] + jnp.dot(ea, w2[...], preferred_element_type=F32)
    msg_o[...] = _ln_relu(v, gm[...], bem[...])


def _msg_tc(ef, ar, bc, we, be_, ge, bee, w2, gm, bem, seg_off):
    d = ef.shape[1]
    e = ar.shape[0]
    blk = 1280
    grid = e // blk
    sb = seg_off // blk
    full = lambda i: (0, 0)
    chunk = lambda i: (i, 0)
    ef_chunk = lambda i: (sb + i, 0)
    return pl.pallas_call(
        _msg_body,
        grid=(grid,),
        in_specs=[
            pl.BlockSpec((blk, d), ef_chunk),
            pl.BlockSpec((blk, H), chunk),
            pl.BlockSpec((blk, H), chunk),
            pl.BlockSpec((d, H), full),
            pl.BlockSpec((1, H), full),
            pl.BlockSpec((1, H), full),
            pl.BlockSpec((1, H), full),
            pl.BlockSpec((H, H), full),
            pl.BlockSpec((1, H), full),
            pl.BlockSpec((1, H), full),
        ],
        out_specs=pl.BlockSpec((blk, H), chunk),
        out_shape=jax.ShapeDtypeStruct((e, H), F32),
    )(ef, ar, bc, we, be_, ge, bee, w2, gm, bem)


def _sc_scatter_kernel(n_nodes, seg_off, seg_e):
    K_EDGE = K_SCATTER
    epw = seg_e // NW
    chunks = epw // K_EDGE
    n_pad = ((n_nodes + NS * 8 - 1) // (NS * 8)) * (NS * 8)
    rpt = n_pad // NS
    mesh = plsc.VectorSubcoreMesh(core_axis_name="c", subcore_axis_name="s")

    @functools.partial(
        pl.kernel,
        mesh=mesh,
        out_type=jax.ShapeDtypeStruct((NC, n_pad, H), F32),
        scratch_types=[
            pltpu.VMEM_SHARED((n_pad, H), F32),
            pltpu.VMEM((epw,), jnp.int32),
            pltpu.VMEM((epw,), jnp.int32),
            pltpu.VMEM((K_EDGE, H), F32),
            pltpu.VMEM((K_EDGE, H), F32),
            pltpu.SemaphoreType.DMA,
            pltpu.SemaphoreType.DMA,
            pltpu.SemaphoreType.DMA,
            pltpu.SemaphoreType.DMA,
            pltpu.SemaphoreType.DMA,
            pltpu.SemaphoreType.DMA,
        ],
    )
    def sc_scatter(msg_hbm, row_hbm, col_hbm, zeros_hbm, out_hbm, aggr_sh,
                   idx_r_all, idx_c_all, m_0, m_1,
                   sl0, sl1, sr0, sr1, sc0, sc1):
        ci = lax.axis_index("c")
        si = lax.axis_index("s")
        wid = ci * NS + si
        pltpu.sync_copy(zeros_hbm.at[pl.ds(si * rpt, rpt)],
                        aggr_sh.at[pl.ds(si * rpt, rpt)])

        base_w = wid * epw
        base_i = seg_off + base_w
        pltpu.sync_copy(row_hbm.at[pl.ds(base_i, epw)], idx_r_all)
        pltpu.sync_copy(col_hbm.at[pl.ds(base_i, epw)], idx_c_all)
        plsc.subcore_barrier()

        sets = ((m_0, sl0, sr0, sc0), (m_1, sl1, sr1, sc1))

        def idx(i, all_):
            return all_.at[pl.ds(i * K_EDGE, K_EDGE)]

        def issue(i, s):
            buf, sl, sr, sc_ = s
            base = base_w + i * K_EDGE
            pltpu.async_copy(msg_hbm.at[pl.ds(base, K_EDGE)], buf, sl)

        def wait_load(i, s):
            buf, sl, sr, sc_ = s
            base = base_w + i * K_EDGE
            pltpu.make_async_copy(
                msg_hbm.at[pl.ds(base, K_EDGE)], buf, sl).wait()

        def scatter(i, s):
            buf, sl, sr, sc_ = s
            pltpu.async_copy(buf, aggr_sh.at[idx(i, idx_r_all)], sr, add=True)
            pltpu.async_copy(buf, aggr_sh.at[idx(i, idx_c_all)], sc_,
                             add=True)

        def wait_scatter(i, s):
            buf, sl, sr, sc_ = s
            pltpu.make_async_copy(buf, aggr_sh.at[idx(i, idx_r_all)],
                                  sr).wait()
            pltpu.make_async_copy(buf, aggr_sh.at[idx(i, idx_c_all)],
                                  sc_).wait()

        issue(0, sets[0])

        def pipe(i, carry):
            wait_load(2 * i, sets[0])
            issue(2 * i + 1, sets[1])
            scatter(2 * i, sets[0])
            wait_load(2 * i + 1, sets[1])
            wait_scatter(2 * i, sets[0])
            issue(2 * i + 2, sets[0])
            scatter(2 * i + 1, sets[1])
            wait_scatter(2 * i + 1, sets[1])
            return carry

        if chunks % 2 == 1:
            lax.fori_loop(0, (chunks - 1) // 2, pipe, 0)
            wait_load(chunks - 1, sets[0])
            scatter(chunks - 1, sets[0])
            wait_scatter(chunks - 1, sets[0])
        else:
            c0 = chunks - 2
            lax.fori_loop(0, (chunks - 2) // 2, pipe, 0)
            wait_load(c0, sets[0])
            issue(c0 + 1, sets[1])
            scatter(c0, sets[0])
            wait_load(c0 + 1, sets[1])
            wait_scatter(c0, sets[0])
            scatter(c0 + 1, sets[1])
            wait_scatter(c0 + 1, sets[1])
        plsc.subcore_barrier()
        pltpu.sync_copy(aggr_sh.at[pl.ds(si * rpt, rpt)],
                        out_hbm.at[ci, pl.ds(si * rpt, rpt)])

    return sc_scatter


def _finish_body(x, p0, p1, p2, p3, bt, u1, u2, bu, gu, beu, wo1, bo1, go,
                 beo, wo2, bo2, out, sums, counts):
    i = pl.program_id(0)
    nsteps = pl.num_programs(0)

    @pl.when(i == 0)
    def _init():
        sums[...] = jnp.zeros_like(sums)
        counts[...] = jnp.zeros_like(counts)

    ag = p0[...] + p1[...] + p2[...] + p3[...]
    h = (jnp.dot(x[...], u1[...], preferred_element_type=F32)
         + jnp.dot(ag, u2[...], preferred_element_type=F32) + bu[...])
    upd = _ln_relu(h, gu[...], beu[...])
    b = bt[...]
    for g in range(4):
        m = b == g
        sums[g:g + 1, :] += jnp.sum(jnp.where(m, upd, 0.0), axis=0,
                                    keepdims=True)
        counts[g:g + 1, :] += jnp.sum(m.astype(F32), axis=0, keepdims=True)

    @pl.when(i == nsteps - 1)
    def _tail():
        rep = sums[...] / jnp.maximum(counts[...], 1.0)
        hh = jnp.dot(rep, wo1[...], preferred_element_type=F32) + bo1[...]
        h2 = _ln_relu(hh, go[...], beo[...])
        o8 = jnp.dot(h2, wo2[...], preferred_element_type=F32) + bo2[...]
        out[...] = o8[0:4, :]


def _finish(x, p0, p1, p2, p3, bt, u1, u2, bu, gu, beu, wo1, bo1, go, beo,
            wo2, bo2):
    n = x.shape[0]
    blk = 1000
    grid = n // blk
    full = lambda i: (0, 0)
    chunk = lambda i: (i, 0)
    return pl.pallas_call(
        _finish_body,
        grid=(grid,),
        in_specs=[
            pl.BlockSpec((blk, H), chunk),
            pl.BlockSpec((blk, H), chunk),
            pl.BlockSpec((blk, H), chunk),
            pl.BlockSpec((blk, H), chunk),
            pl.BlockSpec((blk, H), chunk),
            pl.BlockSpec((blk, 1), chunk),
            pl.BlockSpec((H, H), full),
            pl.BlockSpec((H, H), full),
            pl.BlockSpec((1, H), full),
            pl.BlockSpec((1, H), full),
            pl.BlockSpec((1, H), full),
            pl.BlockSpec((H, H), full),
            pl.BlockSpec((1, H), full),
            pl.BlockSpec((1, H), full),
            pl.BlockSpec((1, H), full),
            pl.BlockSpec((H, H), full),
            pl.BlockSpec((1, H), full),
        ],
        out_specs=pl.BlockSpec((4, H), full),
        out_shape=jax.ShapeDtypeStruct((4, H), F32),
        scratch_shapes=[
            pltpu.VMEM((8, H), F32),
            pltpu.VMEM((8, H), F32),
        ],
    )(x, p0, p1, p2, p3, bt, u1, u2, bu, gu, beu, wo1, bo1, go, beo, wo2,
      bo2)


def kernel(node_features, edge_index, edge_features, edge_types,
           node_positions, batch, is_mutation,
           W_node, b_node, g_node, be_node, W_edge, b_edge, g_edge, be_edge,
           W_msg, b_msg, g_msg, be_msg, W_upd, b_upd, g_upd, be_upd,
           W_o1, b_o1, g_o, be_o, W_o2, b_o2):
    n = node_features.shape[0]
    e = edge_features.shape[0]
    row = edge_index[0].astype(jnp.int32)
    col = edge_index[1].astype(jnp.int32)
    w1 = W_msg[:H]
    w2 = W_msg[H:]
    r2 = lambda v: v.reshape(1, H)

    x, a, b = _node_pre(node_features, W_node, r2(b_node), r2(g_node),
                        r2(be_node), w1, w2, r2(b_msg))
    n_pad = ((n + NS * 8 - 1) // (NS * 8)) * (NS * 8)
    pad = lambda v: jnp.concatenate(
        [v, jnp.zeros((n_pad - n, H), F32)], axis=0)
    a_p, b_p = pad(a), pad(b)
    zeros = jnp.zeros((n_pad, H), F32)
    gran = NW * 80
    es0 = ((e // SEG + gran - 1) // gran) * gran
    seg_sizes = [es0, e - es0]
    seg_offs = [0, es0]
    parts = []
    for s in range(SEG):
        off, es = seg_offs[s], seg_sizes[s]
        ar, bc = _sc_gather_kernel(n, off, es)(a_p, b_p, row, col)
        msg = _msg_tc(edge_features, ar, bc, W_edge, r2(b_edge),
                      r2(g_edge), r2(be_edge), w2, r2(g_msg), r2(be_msg),
                      off)
        parts.append(_sc_scatter_kernel(n, off, es)(msg, row, col, zeros))
    out = _finish(x, parts[0][0, :n], parts[0][1, :n],
                  parts[1][0, :n], parts[1][1, :n],
                  batch.astype(jnp.int32).reshape(n, 1),
                  W_upd[:H], W_upd[H:], r2(b_upd), r2(g_upd), r2(be_upd),
                  W_o1, r2(b_o1), r2(g_o), r2(be_o), W_o2, r2(b_o2))
    return out


# speedup vs baseline: 1.1127x; 1.1127x over previous
import functools

import jax
import jax.numpy as jnp
from jax import lax
from jax.experimental import pallas as pl
from jax.experimental.pallas import tpu as pltpu
from jax.experimental.pallas import tpu_sc as plsc

H = 128
EPS = 1e-5
NC = 2
NS = 16
NW = NC * NS
K_GATHER = 40
K_SCATTER = 80
SEG = 2

F32 = jnp.float32


def _ln_relu(h, g, b):
    mu = jnp.mean(h, axis=-1, keepdims=True)
    var = jnp.mean((h - mu) ** 2, axis=-1, keepdims=True)
    return jnp.maximum((h - mu) * lax.rsqrt(var + EPS) * g + b, 0.0)


def _node_pre_body(nf, wn, bn, gn, ben, w1, w2, bm, x_o, a_o, b_o):
    h = jnp.dot(nf[...], wn[...], preferred_element_type=F32) + bn[...]
    x = _ln_relu(h, gn[...], ben[...])
    x_o[...] = x
    a_o[...] = jnp.dot(x, w1[...], preferred_element_type=F32) + bm[...]
    b_o[...] = jnp.dot(x, w2[...], preferred_element_type=F32)


def _node_pre(nf, wn, bn, gn, ben, w1, w2, bm):
    n = nf.shape[0]
    blk = 2000
    grid = n // blk
    full = lambda i: (0, 0)
    chunk = lambda i: (i, 0)
    specs = [
        pl.BlockSpec((blk, H), chunk),
        pl.BlockSpec((H, H), full),
        pl.BlockSpec((1, H), full),
        pl.BlockSpec((1, H), full),
        pl.BlockSpec((1, H), full),
        pl.BlockSpec((H, H), full),
        pl.BlockSpec((H, H), full),
        pl.BlockSpec((1, H), full),
    ]
    out = jax.ShapeDtypeStruct((n, H), F32)
    return pl.pallas_call(
        _node_pre_body,
        grid=(grid,),
        in_specs=specs,
        out_specs=[pl.BlockSpec((blk, H), chunk)] * 3,
        out_shape=[out, out, out],
    )(nf, wn, bn, gn, ben, w1, w2, bm)


def _sc_gather_kernel(n_nodes, seg_off, seg_e):
    K_EDGE = K_GATHER
    epw = seg_e // NW
    chunks = epw // K_EDGE
    S = 4
    n_pad = ((n_nodes + NS * 8 - 1) // (NS * 8)) * (NS * 8)
    rpt = n_pad // NS
    mesh = plsc.VectorSubcoreMesh(core_axis_name="c", subcore_axis_name="s")
    out = jax.ShapeDtypeStruct((seg_e, H), F32)

    @functools.partial(
        pl.kernel,
        mesh=mesh,
        out_type=[out, out],
        scratch_types=(
            [pltpu.VMEM_SHARED((n_pad, H), F32)]
            + [pltpu.VMEM((epw,), jnp.int32) for _ in range(2)]
            + [pltpu.VMEM((K_EDGE, H), F32) for _ in range(S)]
            + [pltpu.SemaphoreType.DMA for _ in range(2 * S)]
        ),
    )
    def sc_gather(a_hbm, b_hbm, row_hbm, col_hbm, ar_hbm, bc_hbm, *scr):
        tab_sh = scr[0]
        idx_r_all, idx_c_all = scr[1:3]
        buf = scr[3:3 + S]
        sg = scr[3 + S:3 + 2 * S]
        sw = scr[3 + 2 * S:3 + 3 * S]
        ci = lax.axis_index("c")
        si = lax.axis_index("s")
        base_w = (ci * NS + si) * epw
        base_i = seg_off + base_w
        pltpu.sync_copy(row_hbm.at[pl.ds(base_i, epw)], idx_r_all)
        pltpu.sync_copy(col_hbm.at[pl.ds(base_i, epw)], idx_c_all)

        def one_pass(tab_hbm, idx_all, out_hbm):
            pltpu.sync_copy(tab_hbm.at[pl.ds(si * rpt, rpt)],
                            tab_sh.at[pl.ds(si * rpt, rpt)])
            plsc.subcore_barrier()

            def issue_g(i, s):
                pltpu.async_copy(
                    tab_sh.at[idx_all.at[pl.ds(i * K_EDGE, K_EDGE)]],
                    buf[s], sg[s])

            def wait_g(i, s):
                pltpu.make_async_copy(
                    tab_sh.at[idx_all.at[pl.ds(i * K_EDGE, K_EDGE)]],
                    buf[s], sg[s]).wait()

            def issue_w(i, s):
                base = base_w + i * K_EDGE
                pltpu.async_copy(buf[s], out_hbm.at[pl.ds(base, K_EDGE)],
                                 sw[s])

            def wait_w(i, s):
                base = base_w + i * K_EDGE
                pltpu.make_async_copy(
                    buf[s], out_hbm.at[pl.ds(base, K_EDGE)], sw[s]).wait()

            assert chunks >= 6
            issue_g(0, 0)
            issue_g(1, 1)
            issue_g(2, 2)
            wait_g(0, 0)
            issue_w(0, 0)
            issue_g(3, 3)
            wait_g(1, 1)
            issue_w(1, 1)
            wait_w(0, 0)
            issue_g(4, 0)

            def body(i, carry):
                for k in range(S):
                    j = 4 * i + 2 + k
                    s = (2 + k) % S
                    sp = (s + 3) % S
                    wait_g(j, s)
                    issue_w(j, s)
                    wait_w(j - 1, sp)
                    issue_g(j + 3, sp)
                return carry

            niter = (chunks - 5) // 4
            lax.fori_loop(0, niter, body, 0)
            for t in range((chunks - 5) % 4):
                j = 2 + 4 * niter + t
                s = j % S
                sp = (j + 3) % S
                wait_g(j, s)
                issue_w(j, s)
                wait_w(j - 1, sp)
                issue_g(j + 3, sp)
            for j in range(chunks - 3, chunks):
                wait_g(j, j % S)
                issue_w(j, j % S)
            for j in range(chunks - 4, chunks):
                wait_w(j, j % S)
            plsc.subcore_barrier()

        one_pass(a_hbm, idx_r_all, ar_hbm)
        one_pass(b_hbm, idx_c_all, bc_hbm)

    return sc_gather


def _msg_body(ef, ar, bc, we, be_, ge, bee, w2, gm, bem, msg_o):
    h = jnp.dot(ef[...], we[...], preferred_element_type=F32) + be_[...]
    ea = _ln_relu(h, ge[...], bee[...])
    v = ar[...] + bc[...] + jnp.dot(ea, w2[...], preferred_element_type=F32)
    msg_o[...] = _ln_relu(v, gm[...], bem[...])


def _msg_tc(ef, ar, bc, we, be_, ge, bee, w2, gm, bem, seg_off):
    d = ef.shape[1]
    e = ar.shape[0]
    blk = 2560
    grid = e // blk
    sb = seg_off // blk
    full = lambda i: (0, 0)
    chunk = lambda i: (i, 0)
    ef_chunk = lambda i: (sb + i, 0)
    return pl.pallas_call(
        _msg_body,
        grid=(grid,),
        in_specs=[
            pl.BlockSpec((blk, d), ef_chunk),
            pl.BlockSpec((blk, H), chunk),
            pl.BlockSpec((blk, H), chunk),
            pl.BlockSpec((d, H), full),
            pl.BlockSpec((1, H), full),
            pl.BlockSpec((1, H), full),
            pl.BlockSpec((1, H), full),
            pl.BlockSpec((H, H), full),
            pl.BlockSpec((1, H), full),
            pl.BlockSpec((1, H), full),
        ],
        out_specs=pl.BlockSpec((blk, H), chunk),
        out_shape=jax.ShapeDtypeStruct((e, H), F32),
    )(ef, ar, bc, we, be_, ge, bee, w2, gm, bem)


def _sc_scatter_kernel(n_nodes, seg_off, seg_e):
    K_EDGE = K_SCATTER
    epw = seg_e // NW
    chunks = epw // K_EDGE
    n_pad = ((n_nodes + NS * 8 - 1) // (NS * 8)) * (NS * 8)
    rpt = n_pad // NS
    mesh = plsc.VectorSubcoreMesh(core_axis_name="c", subcore_axis_name="s")

    @functools.partial(
        pl.kernel,
        mesh=mesh,
        out_type=jax.ShapeDtypeStruct((NC, n_pad, H), F32),
        scratch_types=[
            pltpu.VMEM_SHARED((n_pad, H), F32),
            pltpu.VMEM((epw,), jnp.int32),
            pltpu.VMEM((epw,), jnp.int32),
            pltpu.VMEM((K_EDGE, H), F32),
            pltpu.VMEM((K_EDGE, H), F32),
            pltpu.SemaphoreType.DMA,
            pltpu.SemaphoreType.DMA,
            pltpu.SemaphoreType.DMA,
            pltpu.SemaphoreType.DMA,
            pltpu.SemaphoreType.DMA,
            pltpu.SemaphoreType.DMA,
        ],
    )
    def sc_scatter(msg_hbm, row_hbm, col_hbm, zeros_hbm, out_hbm, aggr_sh,
                   idx_r_all, idx_c_all, m_0, m_1,
                   sl0, sl1, sr0, sr1, sc0, sc1):
        ci = lax.axis_index("c")
        si = lax.axis_index("s")
        wid = ci * NS + si
        pltpu.sync_copy(zeros_hbm.at[pl.ds(si * rpt, rpt)],
                        aggr_sh.at[pl.ds(si * rpt, rpt)])

        base_w = wid * epw
        base_i = seg_off + base_w
        pltpu.sync_copy(row_hbm.at[pl.ds(base_i, epw)], idx_r_all)
        pltpu.sync_copy(col_hbm.at[pl.ds(base_i, epw)], idx_c_all)
        plsc.subcore_barrier()

        sets = ((m_0, sl0, sr0, sc0), (m_1, sl1, sr1, sc1))

        def idx(i, all_):
            return all_.at[pl.ds(i * K_EDGE, K_EDGE)]

        def issue(i, s):
            buf, sl, sr, sc_ = s
            base = base_w + i * K_EDGE
            pltpu.async_copy(msg_hbm.at[pl.ds(base, K_EDGE)], buf, sl)

        def wait_load(i, s):
            buf, sl, sr, sc_ = s
            base = base_w + i * K_EDGE
            pltpu.make_async_copy(
                msg_hbm.at[pl.ds(base, K_EDGE)], buf, sl).wait()

        def scatter(i, s):
            buf, sl, sr, sc_ = s
            pltpu.async_copy(buf, aggr_sh.at[idx(i, idx_r_all)], sr, add=True)
            pltpu.async_copy(buf, aggr_sh.at[idx(i, idx_c_all)], sc_,
                             add=True)

        def wait_scatter(i, s):
            buf, sl, sr, sc_ = s
            pltpu.make_async_copy(buf, aggr_sh.at[idx(i, idx_r_all)],
                                  sr).wait()
            pltpu.make_async_copy(buf, aggr_sh.at[idx(i, idx_c_all)],
                                  sc_).wait()

        issue(0, sets[0])

        def pipe(i, carry):
            wait_load(2 * i, sets[0])
            issue(2 * i + 1, sets[1])
            scatter(2 * i, sets[0])
            wait_load(2 * i + 1, sets[1])
            wait_scatter(2 * i, sets[0])
            issue(2 * i + 2, sets[0])
            scatter(2 * i + 1, sets[1])
            wait_scatter(2 * i + 1, sets[1])
            return carry

        if chunks % 2 == 1:
            lax.fori_loop(0, (chunks - 1) // 2, pipe, 0)
            wait_load(chunks - 1, sets[0])
            scatter(chunks - 1, sets[0])
            wait_scatter(chunks - 1, sets[0])
        else:
            c0 = chunks - 2
            lax.fori_loop(0, (chunks - 2) // 2, pipe, 0)
            wait_load(c0, sets[0])
            issue(c0 + 1, sets[1])
            scatter(c0, sets[0])
            wait_load(c0 + 1, sets[1])
            wait_scatter(c0, sets[0])
            scatter(c0 + 1, sets[1])
            wait_scatter(c0 + 1, sets[1])
        plsc.subcore_barrier()
        pltpu.sync_copy(aggr_sh.at[pl.ds(si * rpt, rpt)],
                        out_hbm.at[ci, pl.ds(si * rpt, rpt)])

    return sc_scatter


def _finish_body(x, p0, p1, p2, p3, bt, u1, u2, bu, gu, beu, wo1, bo1, go,
                 beo, wo2, bo2, out, sums, counts):
    i = pl.program_id(0)
    nsteps = pl.num_programs(0)

    @pl.when(i == 0)
    def _init():
        sums[...] = jnp.zeros_like(sums)
        counts[...] = jnp.zeros_like(counts)

    ag = p0[...] + p1[...] + p2[...] + p3[...]
    h = (jnp.dot(x[...], u1[...], preferred_element_type=F32)
         + jnp.dot(ag, u2[...], preferred_element_type=F32) + bu[...])
    upd = _ln_relu(h, gu[...], beu[...])
    b = bt[...]
    for g in range(4):
        m = b == g
        sums[g:g + 1, :] += jnp.sum(jnp.where(m, upd, 0.0), axis=0,
                                    keepdims=True)
        counts[g:g + 1, :] += jnp.sum(m.astype(F32), axis=0, keepdims=True)

    @pl.when(i == nsteps - 1)
    def _tail():
        rep = sums[...] / jnp.maximum(counts[...], 1.0)
        hh = jnp.dot(rep, wo1[...], preferred_element_type=F32) + bo1[...]
        h2 = _ln_relu(hh, go[...], beo[...])
        o8 = jnp.dot(h2, wo2[...], preferred_element_type=F32) + bo2[...]
        out[...] = o8[0:4, :]


def _finish(x, p0, p1, p2, p3, bt, u1, u2, bu, gu, beu, wo1, bo1, go, beo,
            wo2, bo2):
    n = x.shape[0]
    blk = 1000
    grid = n // blk
    full = lambda i: (0, 0)
    chunk = lambda i: (i, 0)
    return pl.pallas_call(
        _finish_body,
        grid=(grid,),
        in_specs=[
            pl.BlockSpec((blk, H), chunk),
            pl.BlockSpec((blk, H), chunk),
            pl.BlockSpec((blk, H), chunk),
            pl.BlockSpec((blk, H), chunk),
            pl.BlockSpec((blk, H), chunk),
            pl.BlockSpec((blk, 1), chunk),
            pl.BlockSpec((H, H), full),
            pl.BlockSpec((H, H), full),
            pl.BlockSpec((1, H), full),
            pl.BlockSpec((1, H), full),
            pl.BlockSpec((1, H), full),
            pl.BlockSpec((H, H), full),
            pl.BlockSpec((1, H), full),
            pl.BlockSpec((1, H), full),
            pl.BlockSpec((1, H), full),
            pl.BlockSpec((H, H), full),
            pl.BlockSpec((1, H), full),
        ],
        out_specs=pl.BlockSpec((4, H), full),
        out_shape=jax.ShapeDtypeStruct((4, H), F32),
        scratch_shapes=[
            pltpu.VMEM((8, H), F32),
            pltpu.VMEM((8, H), F32),
        ],
    )(x, p0, p1, p2, p3, bt, u1, u2, bu, gu, beu, wo1, bo1, go, beo, wo2,
      bo2)


def kernel(node_features, edge_index, edge_features, edge_types,
           node_positions, batch, is_mutation,
           W_node, b_node, g_node, be_node, W_edge, b_edge, g_edge, be_edge,
           W_msg, b_msg, g_msg, be_msg, W_upd, b_upd, g_upd, be_upd,
           W_o1, b_o1, g_o, be_o, W_o2, b_o2):
    n = node_features.shape[0]
    e = edge_features.shape[0]
    row = edge_index[0].astype(jnp.int32)
    col = edge_index[1].astype(jnp.int32)
    w1 = W_msg[:H]
    w2 = W_msg[H:]
    r2 = lambda v: v.reshape(1, H)

    x, a, b = _node_pre(node_features, W_node, r2(b_node), r2(g_node),
                        r2(be_node), w1, w2, r2(b_msg))
    n_pad = ((n + NS * 8 - 1) // (NS * 8)) * (NS * 8)
    pad = lambda v: jnp.concatenate(
        [v, jnp.zeros((n_pad - n, H), F32)], axis=0)
    a_p, b_p = pad(a), pad(b)
    zeros = jnp.zeros((n_pad, H), F32)
    gran = NW * 80
    es0 = ((e // SEG + gran - 1) // gran) * gran
    seg_sizes = [es0, e - es0]
    seg_offs = [0, es0]
    parts = []
    for s in range(SEG):
        off, es = seg_offs[s], seg_sizes[s]
        ar, bc = _sc_gather_kernel(n, off, es)(a_p, b_p, row, col)
        msg = _msg_tc(edge_features, ar, bc, W_edge, r2(b_edge),
                      r2(g_edge), r2(be_edge), w2, r2(g_msg), r2(be_msg),
                      off)
        parts.append(_sc_scatter_kernel(n, off, es)(msg, row, col, zeros))
    out = _finish(x, parts[0][0, :n], parts[0][1, :n],
                  parts[1][0, :n], parts[1][1, :n],
                  batch.astype(jnp.int32).reshape(n, 1),
                  W_upd[:H], W_upd[H:], r2(b_upd), r2(g_upd), r2(be_upd),
                  W_o1, r2(b_o1), r2(g_o), r2(be_o), W_o2, r2(b_o2))
    return out


# speedup vs baseline: 1.1478x; 1.0316x over previous
import functools

import jax
import jax.numpy as jnp
from jax import lax
from jax.experimental import pallas as pl
from jax.experimental.pallas import tpu as pltpu
from jax.experimental.pallas import tpu_sc as plsc

H = 128
EPS = 1e-5
NC = 2
NS = 16
NW = NC * NS
K_GATHER = 40
K_SCATTER = 80
SEG = 2

F32 = jnp.float32


def _ln_relu(h, g, b):
    mu = jnp.mean(h, axis=-1, keepdims=True)
    var = jnp.mean((h - mu) ** 2, axis=-1, keepdims=True)
    return jnp.maximum((h - mu) * lax.rsqrt(var + EPS) * g + b, 0.0)


def _node_pre_body(nf, wn, bn, gn, ben, w1, w2, bm, x_o, a_o, b_o):
    h = jnp.dot(nf[...], wn[...], preferred_element_type=F32) + bn[...]
    x = _ln_relu(h, gn[...], ben[...])
    x_o[...] = x
    a_o[...] = jnp.dot(x, w1[...], preferred_element_type=F32) + bm[...]
    b_o[...] = jnp.dot(x, w2[...], preferred_element_type=F32)


def _node_pre(nf, wn, bn, gn, ben, w1, w2, bm):
    n = nf.shape[0]
    blk = 2000
    grid = n // blk
    full = lambda i: (0, 0)
    chunk = lambda i: (i, 0)
    specs = [
        pl.BlockSpec((blk, H), chunk),
        pl.BlockSpec((H, H), full),
        pl.BlockSpec((1, H), full),
        pl.BlockSpec((1, H), full),
        pl.BlockSpec((1, H), full),
        pl.BlockSpec((H, H), full),
        pl.BlockSpec((H, H), full),
        pl.BlockSpec((1, H), full),
    ]
    out = jax.ShapeDtypeStruct((n, H), F32)
    return pl.pallas_call(
        _node_pre_body,
        grid=(grid,),
        in_specs=specs,
        out_specs=[pl.BlockSpec((blk, H), chunk)] * 3,
        out_shape=[out, out, out],
    )(nf, wn, bn, gn, ben, w1, w2, bm)


def _sc_gather_kernel(n_nodes, seg_off, seg_e):
    K_EDGE = K_GATHER
    epw = seg_e // NW
    chunks = epw // K_EDGE
    S = 4
    n_pad = ((n_nodes + NS * 8 - 1) // (NS * 8)) * (NS * 8)
    rpt = n_pad // NS
    mesh = plsc.VectorSubcoreMesh(core_axis_name="c", subcore_axis_name="s")
    out = jax.ShapeDtypeStruct((seg_e, H), F32)

    @functools.partial(
        pl.kernel,
        mesh=mesh,
        out_type=[out, out],
        scratch_types=(
            [pltpu.VMEM_SHARED((n_pad, H), F32)]
            + [pltpu.VMEM((epw,), jnp.int32) for _ in range(2)]
            + [pltpu.VMEM((K_EDGE, H), F32) for _ in range(S)]
            + [pltpu.SemaphoreType.DMA for _ in range(2 * S)]
        ),
    )
    def sc_gather(a_hbm, b_hbm, row_hbm, col_hbm, ar_hbm, bc_hbm, *scr):
        tab_sh = scr[0]
        idx_r_all, idx_c_all = scr[1:3]
        buf = scr[3:3 + S]
        sg = scr[3 + S:3 + 2 * S]
        sw = scr[3 + 2 * S:3 + 3 * S]
        ci = lax.axis_index("c")
        si = lax.axis_index("s")
        base_w = (ci * NS + si) * epw
        base_i = seg_off + base_w
        pltpu.sync_copy(row_hbm.at[pl.ds(base_i, epw)], idx_r_all)
        pltpu.sync_copy(col_hbm.at[pl.ds(base_i, epw)], idx_c_all)

        def one_pass(tab_hbm, idx_all, out_hbm):
            pltpu.sync_copy(tab_hbm.at[pl.ds(si * rpt, rpt)],
                            tab_sh.at[pl.ds(si * rpt, rpt)])
            plsc.subcore_barrier()

            def issue_g(i, s):
                pltpu.async_copy(
                    tab_sh.at[idx_all.at[pl.ds(i * K_EDGE, K_EDGE)]],
                    buf[s], sg[s])

            def wait_g(i, s):
                pltpu.make_async_copy(
                    tab_sh.at[idx_all.at[pl.ds(i * K_EDGE, K_EDGE)]],
                    buf[s], sg[s]).wait()

            def issue_w(i, s):
                base = base_w + i * K_EDGE
                pltpu.async_copy(buf[s], out_hbm.at[pl.ds(base, K_EDGE)],
                                 sw[s])

            def wait_w(i, s):
                base = base_w + i * K_EDGE
                pltpu.make_async_copy(
                    buf[s], out_hbm.at[pl.ds(base, K_EDGE)], sw[s]).wait()

            assert chunks >= 6
            issue_g(0, 0)
            issue_g(1, 1)
            issue_g(2, 2)
            wait_g(0, 0)
            issue_w(0, 0)
            issue_g(3, 3)
            wait_g(1, 1)
            issue_w(1, 1)
            wait_w(0, 0)
            issue_g(4, 0)

            def body(i, carry):
                for k in range(S):
                    j = 4 * i + 2 + k
                    s = (2 + k) % S
                    sp = (s + 3) % S
                    wait_g(j, s)
                    issue_w(j, s)
                    wait_w(j - 1, sp)
                    issue_g(j + 3, sp)
                return carry

            niter = (chunks - 5) // 4
            lax.fori_loop(0, niter, body, 0)
            for t in range((chunks - 5) % 4):
                j = 2 + 4 * niter + t
                s = j % S
                sp = (j + 3) % S
                wait_g(j, s)
                issue_w(j, s)
                wait_w(j - 1, sp)
                issue_g(j + 3, sp)
            for j in range(chunks - 3, chunks):
                wait_g(j, j % S)
                issue_w(j, j % S)
            for j in range(chunks - 4, chunks):
                wait_w(j, j % S)
            plsc.subcore_barrier()

        one_pass(a_hbm, idx_r_all, ar_hbm)
        one_pass(b_hbm, idx_c_all, bc_hbm)

    return sc_gather


def _msg_body(ef, ar, bc, we, be_, ge, bee, w2, gm, bem, msg_o):
    h = jnp.dot(ef[...], we[...], preferred_element_type=F32) + be_[...]
    ea = _ln_relu(h, ge[...], bee[...])
    v = ar[...] + bc[...] + jnp.dot(ea, w2[...], preferred_element_type=F32)
    msg_o[...] = _ln_relu(v, gm[...], bem[...])


def _msg_tc(ef, ar, bc, we, be_, ge, bee, w2, gm, bem, seg_off):
    d = ef.shape[1]
    e = ar.shape[0]
    blk = 6400
    grid = e // blk
    sb = seg_off // blk
    full = lambda i: (0, 0)
    chunk = lambda i: (i, 0)
    ef_chunk = lambda i: (sb + i, 0)
    return pl.pallas_call(
        _msg_body,
        grid=(grid,),
        in_specs=[
            pl.BlockSpec((blk, d), ef_chunk),
            pl.BlockSpec((blk, H), chunk),
            pl.BlockSpec((blk, H), chunk),
            pl.BlockSpec((d, H), full),
            pl.BlockSpec((1, H), full),
            pl.BlockSpec((1, H), full),
            pl.BlockSpec((1, H), full),
            pl.BlockSpec((H, H), full),
            pl.BlockSpec((1, H), full),
            pl.BlockSpec((1, H), full),
        ],
        out_specs=pl.BlockSpec((blk, H), chunk),
        out_shape=jax.ShapeDtypeStruct((e, H), F32),
    )(ef, ar, bc, we, be_, ge, bee, w2, gm, bem)


def _sc_scatter_kernel(n_nodes, seg_off, seg_e):
    K_EDGE = K_SCATTER
    epw = seg_e // NW
    chunks = epw // K_EDGE
    n_pad = ((n_nodes + NS * 8 - 1) // (NS * 8)) * (NS * 8)
    rpt = n_pad // NS
    mesh = plsc.VectorSubcoreMesh(core_axis_name="c", subcore_axis_name="s")

    @functools.partial(
        pl.kernel,
        mesh=mesh,
        out_type=jax.ShapeDtypeStruct((NC, n_pad, H), F32),
        scratch_types=[
            pltpu.VMEM_SHARED((n_pad, H), F32),
            pltpu.VMEM((epw,), jnp.int32),
            pltpu.VMEM((epw,), jnp.int32),
            pltpu.VMEM((K_EDGE, H), F32),
            pltpu.VMEM((K_EDGE, H), F32),
            pltpu.SemaphoreType.DMA,
            pltpu.SemaphoreType.DMA,
            pltpu.SemaphoreType.DMA,
            pltpu.SemaphoreType.DMA,
            pltpu.SemaphoreType.DMA,
            pltpu.SemaphoreType.DMA,
        ],
    )
    def sc_scatter(msg_hbm, row_hbm, col_hbm, zeros_hbm, out_hbm, aggr_sh,
                   idx_r_all, idx_c_all, m_0, m_1,
                   sl0, sl1, sr0, sr1, sc0, sc1):
        ci = lax.axis_index("c")
        si = lax.axis_index("s")
        wid = ci * NS + si
        pltpu.sync_copy(zeros_hbm.at[pl.ds(si * rpt, rpt)],
                        aggr_sh.at[pl.ds(si * rpt, rpt)])

        base_w = wid * epw
        base_i = seg_off + base_w
        pltpu.sync_copy(row_hbm.at[pl.ds(base_i, epw)], idx_r_all)
        pltpu.sync_copy(col_hbm.at[pl.ds(base_i, epw)], idx_c_all)
        plsc.subcore_barrier()

        sets = ((m_0, sl0, sr0, sc0), (m_1, sl1, sr1, sc1))

        def idx(i, all_):
            return all_.at[pl.ds(i * K_EDGE, K_EDGE)]

        def issue(i, s):
            buf, sl, sr, sc_ = s
            base = base_w + i * K_EDGE
            pltpu.async_copy(msg_hbm.at[pl.ds(base, K_EDGE)], buf, sl)

        def wait_load(i, s):
            buf, sl, sr, sc_ = s
            base = base_w + i * K_EDGE
            pltpu.make_async_copy(
                msg_hbm.at[pl.ds(base, K_EDGE)], buf, sl).wait()

        def scatter(i, s):
            buf, sl, sr, sc_ = s
            pltpu.async_copy(buf, aggr_sh.at[idx(i, idx_r_all)], sr, add=True)
            pltpu.async_copy(buf, aggr_sh.at[idx(i, idx_c_all)], sc_,
                             add=True)

        def wait_scatter(i, s):
            buf, sl, sr, sc_ = s
            pltpu.make_async_copy(buf, aggr_sh.at[idx(i, idx_r_all)],
                                  sr).wait()
            pltpu.make_async_copy(buf, aggr_sh.at[idx(i, idx_c_all)],
                                  sc_).wait()

        issue(0, sets[0])

        def pipe(i, carry):
            wait_load(2 * i, sets[0])
            issue(2 * i + 1, sets[1])
            scatter(2 * i, sets[0])
            wait_load(2 * i + 1, sets[1])
            wait_scatter(2 * i, sets[0])
            issue(2 * i + 2, sets[0])
            scatter(2 * i + 1, sets[1])
            wait_scatter(2 * i + 1, sets[1])
            return carry

        if chunks % 2 == 1:
            lax.fori_loop(0, (chunks - 1) // 2, pipe, 0)
            wait_load(chunks - 1, sets[0])
            scatter(chunks - 1, sets[0])
            wait_scatter(chunks - 1, sets[0])
        else:
            c0 = chunks - 2
            lax.fori_loop(0, (chunks - 2) // 2, pipe, 0)
            wait_load(c0, sets[0])
            issue(c0 + 1, sets[1])
            scatter(c0, sets[0])
            wait_load(c0 + 1, sets[1])
            wait_scatter(c0, sets[0])
            scatter(c0 + 1, sets[1])
            wait_scatter(c0 + 1, sets[1])
        plsc.subcore_barrier()
        pltpu.sync_copy(aggr_sh.at[pl.ds(si * rpt, rpt)],
                        out_hbm.at[ci, pl.ds(si * rpt, rpt)])

    return sc_scatter


def _finish_body(x, p0, p1, p2, p3, bt, u1, u2, bu, gu, beu, wo1, bo1, go,
                 beo, wo2, bo2, out, sums, counts):
    i = pl.program_id(0)
    nsteps = pl.num_programs(0)

    @pl.when(i == 0)
    def _init():
        sums[...] = jnp.zeros_like(sums)
        counts[...] = jnp.zeros_like(counts)

    ag = p0[...] + p1[...] + p2[...] + p3[...]
    h = (jnp.dot(x[...], u1[...], preferred_element_type=F32)
         + jnp.dot(ag, u2[...], preferred_element_type=F32) + bu[...])
    upd = _ln_relu(h, gu[...], beu[...])
    b = bt[...]
    for g in range(4):
        m = b == g
        sums[g:g + 1, :] += jnp.sum(jnp.where(m, upd, 0.0), axis=0,
                                    keepdims=True)
        counts[g:g + 1, :] += jnp.sum(m.astype(F32), axis=0, keepdims=True)

    @pl.when(i == nsteps - 1)
    def _tail():
        rep = sums[...] / jnp.maximum(counts[...], 1.0)
        hh = jnp.dot(rep, wo1[...], preferred_element_type=F32) + bo1[...]
        h2 = _ln_relu(hh, go[...], beo[...])
        o8 = jnp.dot(h2, wo2[...], preferred_element_type=F32) + bo2[...]
        out[...] = o8[0:4, :]


def _finish(x, p0, p1, p2, p3, bt, u1, u2, bu, gu, beu, wo1, bo1, go, beo,
            wo2, bo2):
    n = x.shape[0]
    blk = 1000
    grid = n // blk
    full = lambda i: (0, 0)
    chunk = lambda i: (i, 0)
    return pl.pallas_call(
        _finish_body,
        grid=(grid,),
        in_specs=[
            pl.BlockSpec((blk, H), chunk),
            pl.BlockSpec((blk, H), chunk),
            pl.BlockSpec((blk, H), chunk),
            pl.BlockSpec((blk, H), chunk),
            pl.BlockSpec((blk, H), chunk),
            pl.BlockSpec((blk, 1), chunk),
            pl.BlockSpec((H, H), full),
            pl.BlockSpec((H, H), full),
            pl.BlockSpec((1, H), full),
            pl.BlockSpec((1, H), full),
            pl.BlockSpec((1, H), full),
            pl.BlockSpec((H, H), full),
            pl.BlockSpec((1, H), full),
            pl.BlockSpec((1, H), full),
            pl.BlockSpec((1, H), full),
            pl.BlockSpec((H, H), full),
            pl.BlockSpec((1, H), full),
        ],
        out_specs=pl.BlockSpec((4, H), full),
        out_shape=jax.ShapeDtypeStruct((4, H), F32),
        scratch_shapes=[
            pltpu.VMEM((8, H), F32),
            pltpu.VMEM((8, H), F32),
        ],
    )(x, p0, p1, p2, p3, bt, u1, u2, bu, gu, beu, wo1, bo1, go, beo, wo2,
      bo2)


def kernel(node_features, edge_index, edge_features, edge_types,
           node_positions, batch, is_mutation,
           W_node, b_node, g_node, be_node, W_edge, b_edge, g_edge, be_edge,
           W_msg, b_msg, g_msg, be_msg, W_upd, b_upd, g_upd, be_upd,
           W_o1, b_o1, g_o, be_o, W_o2, b_o2):
    n = node_features.shape[0]
    e = edge_features.shape[0]
    row = edge_index[0].astype(jnp.int32)
    col = edge_index[1].astype(jnp.int32)
    w1 = W_msg[:H]
    w2 = W_msg[H:]
    r2 = lambda v: v.reshape(1, H)

    x, a, b = _node_pre(node_features, W_node, r2(b_node), r2(g_node),
                        r2(be_node), w1, w2, r2(b_msg))
    n_pad = ((n + NS * 8 - 1) // (NS * 8)) * (NS * 8)
    pad = lambda v: jnp.concatenate(
        [v, jnp.zeros((n_pad - n, H), F32)], axis=0)
    a_p, b_p = pad(a), pad(b)
    zeros = jnp.zeros((n_pad, H), F32)
    gran = NW * 80 * 5
    es0 = ((e // SEG + gran - 1) // gran) * gran
    seg_sizes = [es0, e - es0]
    seg_offs = [0, es0]
    parts = []
    for s in range(SEG):
        off, es = seg_offs[s], seg_sizes[s]
        ar, bc = _sc_gather_kernel(n, off, es)(a_p, b_p, row, col)
        msg = _msg_tc(edge_features, ar, bc, W_edge, r2(b_edge),
                      r2(g_edge), r2(be_edge), w2, r2(g_msg), r2(be_msg),
                      off)
        parts.append(_sc_scatter_kernel(n, off, es)(msg, row, col, zeros))
    out = _finish(x, parts[0][0, :n], parts[0][1, :n],
                  parts[1][0, :n], parts[1][1, :n],
                  batch.astype(jnp.int32).reshape(n, 1),
                  W_upd[:H], W_upd[H:], r2(b_upd), r2(g_upd), r2(be_upd),
                  W_o1, r2(b_o1), r2(g_o), r2(be_o), W_o2, r2(b_o2))
    return out


# speedup vs baseline: 1.1606x; 1.0111x over previous
import functools

import jax
import jax.numpy as jnp
from jax import lax
from jax.experimental import pallas as pl
from jax.experimental.pallas import tpu as pltpu
from jax.experimental.pallas import tpu_sc as plsc

H = 128
EPS = 1e-5
NC = 2
NS = 16
NW = NC * NS
K_GATHER = 40
K_SCATTER = 80
SEG = 2

F32 = jnp.float32


def _ln_relu(h, g, b):
    mu = jnp.mean(h, axis=-1, keepdims=True)
    var = jnp.mean((h - mu) ** 2, axis=-1, keepdims=True)
    return jnp.maximum((h - mu) * lax.rsqrt(var + EPS) * g + b, 0.0)


def _node_pre_body(nf, wn, bn, gn, ben, w1, w2, bm, x_o, a_o, b_o):
    h = jnp.dot(nf[...], wn[...], preferred_element_type=F32) + bn[...]
    x = _ln_relu(h, gn[...], ben[...])
    x_o[...] = x
    a_o[...] = jnp.dot(x, w1[...], preferred_element_type=F32) + bm[...]
    b_o[...] = jnp.dot(x, w2[...], preferred_element_type=F32)


def _node_pre(nf, wn, bn, gn, ben, w1, w2, bm):
    n = nf.shape[0]
    blk = 5000
    grid = n // blk
    full = lambda i: (0, 0)
    chunk = lambda i: (i, 0)
    specs = [
        pl.BlockSpec((blk, H), chunk),
        pl.BlockSpec((H, H), full),
        pl.BlockSpec((1, H), full),
        pl.BlockSpec((1, H), full),
        pl.BlockSpec((1, H), full),
        pl.BlockSpec((H, H), full),
        pl.BlockSpec((H, H), full),
        pl.BlockSpec((1, H), full),
    ]
    out = jax.ShapeDtypeStruct((n, H), F32)
    return pl.pallas_call(
        _node_pre_body,
        grid=(grid,),
        in_specs=specs,
        out_specs=[pl.BlockSpec((blk, H), chunk)] * 3,
        out_shape=[out, out, out],
    )(nf, wn, bn, gn, ben, w1, w2, bm)


def _sc_gather_kernel(n_nodes, seg_off, seg_e):
    K_EDGE = K_GATHER
    epw = seg_e // NW
    chunks = epw // K_EDGE
    S = 4
    n_pad = ((n_nodes + NS * 8 - 1) // (NS * 8)) * (NS * 8)
    rpt = n_pad // NS
    mesh = plsc.VectorSubcoreMesh(core_axis_name="c", subcore_axis_name="s")
    out = jax.ShapeDtypeStruct((seg_e, H), F32)

    @functools.partial(
        pl.kernel,
        mesh=mesh,
        out_type=[out, out],
        scratch_types=(
            [pltpu.VMEM_SHARED((n_pad, H), F32)]
            + [pltpu.VMEM((epw,), jnp.int32) for _ in range(2)]
            + [pltpu.VMEM((K_EDGE, H), F32) for _ in range(S)]
            + [pltpu.SemaphoreType.DMA for _ in range(2 * S)]
        ),
    )
    def sc_gather(a_hbm, b_hbm, row_hbm, col_hbm, ar_hbm, bc_hbm, *scr):
        tab_sh = scr[0]
        idx_r_all, idx_c_all = scr[1:3]
        buf = scr[3:3 + S]
        sg = scr[3 + S:3 + 2 * S]
        sw = scr[3 + 2 * S:3 + 3 * S]
        ci = lax.axis_index("c")
        si = lax.axis_index("s")
        base_w = (ci * NS + si) * epw
        base_i = seg_off + base_w
        pltpu.sync_copy(row_hbm.at[pl.ds(base_i, epw)], idx_r_all)
        pltpu.sync_copy(col_hbm.at[pl.ds(base_i, epw)], idx_c_all)

        def one_pass(tab_hbm, idx_all, out_hbm):
            pltpu.sync_copy(tab_hbm.at[pl.ds(si * rpt, rpt)],
                            tab_sh.at[pl.ds(si * rpt, rpt)])
            plsc.subcore_barrier()

            def issue_g(i, s):
                pltpu.async_copy(
                    tab_sh.at[idx_all.at[pl.ds(i * K_EDGE, K_EDGE)]],
                    buf[s], sg[s])

            def wait_g(i, s):
                pltpu.make_async_copy(
                    tab_sh.at[idx_all.at[pl.ds(i * K_EDGE, K_EDGE)]],
                    buf[s], sg[s]).wait()

            def issue_w(i, s):
                base = base_w + i * K_EDGE
                pltpu.async_copy(buf[s], out_hbm.at[pl.ds(base, K_EDGE)],
                                 sw[s])

            def wait_w(i, s):
                base = base_w + i * K_EDGE
                pltpu.make_async_copy(
                    buf[s], out_hbm.at[pl.ds(base, K_EDGE)], sw[s]).wait()

            assert chunks >= 6
            issue_g(0, 0)
            issue_g(1, 1)
            issue_g(2, 2)
            wait_g(0, 0)
            issue_w(0, 0)
            issue_g(3, 3)
            wait_g(1, 1)
            issue_w(1, 1)
            wait_w(0, 0)
            issue_g(4, 0)

            def body(i, carry):
                for k in range(S):
                    j = 4 * i + 2 + k
                    s = (2 + k) % S
                    sp = (s + 3) % S
                    wait_g(j, s)
                    issue_w(j, s)
                    wait_w(j - 1, sp)
                    issue_g(j + 3, sp)
                return carry

            niter = (chunks - 5) // 4
            lax.fori_loop(0, niter, body, 0)
            for t in range((chunks - 5) % 4):
                j = 2 + 4 * niter + t
                s = j % S
                sp = (j + 3) % S
                wait_g(j, s)
                issue_w(j, s)
                wait_w(j - 1, sp)
                issue_g(j + 3, sp)
            for j in range(chunks - 3, chunks):
                wait_g(j, j % S)
                issue_w(j, j % S)
            for j in range(chunks - 4, chunks):
                wait_w(j, j % S)
            plsc.subcore_barrier()

        one_pass(a_hbm, idx_r_all, ar_hbm)
        one_pass(b_hbm, idx_c_all, bc_hbm)

    return sc_gather


def _msg_body(ef, ar, bc, we, be_, ge, bee, w2, gm, bem, msg_o):
    h = jnp.dot(ef[...], we[...], preferred_element_type=F32) + be_[...]
    ea = _ln_relu(h, ge[...], bee[...])
    v = ar[...] + bc[...] + jnp.dot(ea, w2[...], preferred_element_type=F32)
    msg_o[...] = _ln_relu(v, gm[...], bem[...])


def _msg_tc(ef, ar, bc, we, be_, ge, bee, w2, gm, bem, seg_off):
    d = ef.shape[1]
    e = ar.shape[0]
    blk = 6400
    grid = e // blk
    sb = seg_off // blk
    full = lambda i: (0, 0)
    chunk = lambda i: (i, 0)
    ef_chunk = lambda i: (sb + i, 0)
    return pl.pallas_call(
        _msg_body,
        grid=(grid,),
        in_specs=[
            pl.BlockSpec((blk, d), ef_chunk),
            pl.BlockSpec((blk, H), chunk),
            pl.BlockSpec((blk, H), chunk),
            pl.BlockSpec((d, H), full),
            pl.BlockSpec((1, H), full),
            pl.BlockSpec((1, H), full),
            pl.BlockSpec((1, H), full),
            pl.BlockSpec((H, H), full),
            pl.BlockSpec((1, H), full),
            pl.BlockSpec((1, H), full),
        ],
        out_specs=pl.BlockSpec((blk, H), chunk),
        out_shape=jax.ShapeDtypeStruct((e, H), F32),
    )(ef, ar, bc, we, be_, ge, bee, w2, gm, bem)


def _sc_scatter_kernel(n_nodes, seg_off, seg_e):
    K_EDGE = K_SCATTER
    epw = seg_e // NW
    chunks = epw // K_EDGE
    n_pad = ((n_nodes + NS * 8 - 1) // (NS * 8)) * (NS * 8)
    rpt = n_pad // NS
    mesh = plsc.VectorSubcoreMesh(core_axis_name="c", subcore_axis_name="s")

    @functools.partial(
        pl.kernel,
        mesh=mesh,
        out_type=jax.ShapeDtypeStruct((NC, n_pad, H), F32),
        scratch_types=[
            pltpu.VMEM_SHARED((n_pad, H), F32),
            pltpu.VMEM((epw,), jnp.int32),
            pltpu.VMEM((epw,), jnp.int32),
            pltpu.VMEM((K_EDGE, H), F32),
            pltpu.VMEM((K_EDGE, H), F32),
            pltpu.SemaphoreType.DMA,
            pltpu.SemaphoreType.DMA,
            pltpu.SemaphoreType.DMA,
            pltpu.SemaphoreType.DMA,
            pltpu.SemaphoreType.DMA,
            pltpu.SemaphoreType.DMA,
        ],
    )
    def sc_scatter(msg_hbm, row_hbm, col_hbm, zeros_hbm, out_hbm, aggr_sh,
                   idx_r_all, idx_c_all, m_0, m_1,
                   sl0, sl1, sr0, sr1, sc0, sc1):
        ci = lax.axis_index("c")
        si = lax.axis_index("s")
        wid = ci * NS + si
        pltpu.sync_copy(zeros_hbm.at[pl.ds(si * rpt, rpt)],
                        aggr_sh.at[pl.ds(si * rpt, rpt)])

        base_w = wid * epw
        base_i = seg_off + base_w
        pltpu.sync_copy(row_hbm.at[pl.ds(base_i, epw)], idx_r_all)
        pltpu.sync_copy(col_hbm.at[pl.ds(base_i, epw)], idx_c_all)
        plsc.subcore_barrier()

        sets = ((m_0, sl0, sr0, sc0), (m_1, sl1, sr1, sc1))

        def idx(i, all_):
            return all_.at[pl.ds(i * K_EDGE, K_EDGE)]

        def issue(i, s):
            buf, sl, sr, sc_ = s
            base = base_w + i * K_EDGE
            pltpu.async_copy(msg_hbm.at[pl.ds(base, K_EDGE)], buf, sl)

        def wait_load(i, s):
            buf, sl, sr, sc_ = s
            base = base_w + i * K_EDGE
            pltpu.make_async_copy(
                msg_hbm.at[pl.ds(base, K_EDGE)], buf, sl).wait()

        def scatter(i, s):
            buf, sl, sr, sc_ = s
            pltpu.async_copy(buf, aggr_sh.at[idx(i, idx_r_all)], sr, add=True)
            pltpu.async_copy(buf, aggr_sh.at[idx(i, idx_c_all)], sc_,
                             add=True)

        def wait_scatter(i, s):
            buf, sl, sr, sc_ = s
            pltpu.make_async_copy(buf, aggr_sh.at[idx(i, idx_r_all)],
                                  sr).wait()
            pltpu.make_async_copy(buf, aggr_sh.at[idx(i, idx_c_all)],
                                  sc_).wait()

        issue(0, sets[0])

        def pipe(i, carry):
            wait_load(2 * i, sets[0])
            issue(2 * i + 1, sets[1])
            scatter(2 * i, sets[0])
            wait_load(2 * i + 1, sets[1])
            wait_scatter(2 * i, sets[0])
            issue(2 * i + 2, sets[0])
            scatter(2 * i + 1, sets[1])
            wait_scatter(2 * i + 1, sets[1])
            return carry

        if chunks % 2 == 1:
            lax.fori_loop(0, (chunks - 1) // 2, pipe, 0)
            wait_load(chunks - 1, sets[0])
            scatter(chunks - 1, sets[0])
            wait_scatter(chunks - 1, sets[0])
        else:
            c0 = chunks - 2
            lax.fori_loop(0, (chunks - 2) // 2, pipe, 0)
            wait_load(c0, sets[0])
            issue(c0 + 1, sets[1])
            scatter(c0, sets[0])
            wait_load(c0 + 1, sets[1])
            wait_scatter(c0, sets[0])
            scatter(c0 + 1, sets[1])
            wait_scatter(c0 + 1, sets[1])
        plsc.subcore_barrier()
        pltpu.sync_copy(aggr_sh.at[pl.ds(si * rpt, rpt)],
                        out_hbm.at[ci, pl.ds(si * rpt, rpt)])

    return sc_scatter


def _finish_body(x, p0, p1, p2, p3, bt, u1, u2, bu, gu, beu, wo1, bo1, go,
                 beo, wo2, bo2, out, sums, counts):
    i = pl.program_id(0)
    nsteps = pl.num_programs(0)

    @pl.when(i == 0)
    def _init():
        sums[...] = jnp.zeros_like(sums)
        counts[...] = jnp.zeros_like(counts)

    ag = p0[...] + p1[...] + p2[...] + p3[...]
    h = (jnp.dot(x[...], u1[...], preferred_element_type=F32)
         + jnp.dot(ag, u2[...], preferred_element_type=F32) + bu[...])
    upd = _ln_relu(h, gu[...], beu[...])
    b = bt[...]
    for g in range(4):
        m = b == g
        sums[g:g + 1, :] += jnp.sum(jnp.where(m, upd, 0.0), axis=0,
                                    keepdims=True)
        counts[g:g + 1, :] += jnp.sum(m.astype(F32), axis=0, keepdims=True)

    @pl.when(i == nsteps - 1)
    def _tail():
        rep = sums[...] / jnp.maximum(counts[...], 1.0)
        hh = jnp.dot(rep, wo1[...], preferred_element_type=F32) + bo1[...]
        h2 = _ln_relu(hh, go[...], beo[...])
        o8 = jnp.dot(h2, wo2[...], preferred_element_type=F32) + bo2[...]
        out[...] = o8[0:4, :]


def _finish(x, p0, p1, p2, p3, bt, u1, u2, bu, gu, beu, wo1, bo1, go, beo,
            wo2, bo2):
    n = x.shape[0]
    blk = 2000
    grid = n // blk
    full = lambda i: (0, 0)
    chunk = lambda i: (i, 0)
    return pl.pallas_call(
        _finish_body,
        grid=(grid,),
        in_specs=[
            pl.BlockSpec((blk, H), chunk),
            pl.BlockSpec((blk, H), chunk),
            pl.BlockSpec((blk, H), chunk),
            pl.BlockSpec((blk, H), chunk),
            pl.BlockSpec((blk, H), chunk),
            pl.BlockSpec((blk, 1), chunk),
            pl.BlockSpec((H, H), full),
            pl.BlockSpec((H, H), full),
            pl.BlockSpec((1, H), full),
            pl.BlockSpec((1, H), full),
            pl.BlockSpec((1, H), full),
            pl.BlockSpec((H, H), full),
            pl.BlockSpec((1, H), full),
            pl.BlockSpec((1, H), full),
            pl.BlockSpec((1, H), full),
            pl.BlockSpec((H, H), full),
            pl.BlockSpec((1, H), full),
        ],
        out_specs=pl.BlockSpec((4, H), full),
        out_shape=jax.ShapeDtypeStruct((4, H), F32),
        scratch_shapes=[
            pltpu.VMEM((8, H), F32),
            pltpu.VMEM((8, H), F32),
        ],
    )(x, p0, p1, p2, p3, bt, u1, u2, bu, gu, beu, wo1, bo1, go, beo, wo2,
      bo2)


def kernel(node_features, edge_index, edge_features, edge_types,
           node_positions, batch, is_mutation,
           W_node, b_node, g_node, be_node, W_edge, b_edge, g_edge, be_edge,
           W_msg, b_msg, g_msg, be_msg, W_upd, b_upd, g_upd, be_upd,
           W_o1, b_o1, g_o, be_o, W_o2, b_o2):
    n = node_features.shape[0]
    e = edge_features.shape[0]
    row = edge_index[0].astype(jnp.int32)
    col = edge_index[1].astype(jnp.int32)
    w1 = W_msg[:H]
    w2 = W_msg[H:]
    r2 = lambda v: v.reshape(1, H)

    x, a, b = _node_pre(node_features, W_node, r2(b_node), r2(g_node),
                        r2(be_node), w1, w2, r2(b_msg))
    n_pad = ((n + NS * 8 - 1) // (NS * 8)) * (NS * 8)
    pad = lambda v: jnp.concatenate(
        [v, jnp.zeros((n_pad - n, H), F32)], axis=0)
    a_p, b_p = pad(a), pad(b)
    zeros = jnp.zeros((n_pad, H), F32)
    gran = NW * 80 * 5
    es0 = ((e // SEG + gran - 1) // gran) * gran
    seg_sizes = [es0, e - es0]
    seg_offs = [0, es0]
    parts = []
    for s in range(SEG):
        off, es = seg_offs[s], seg_sizes[s]
        ar, bc = _sc_gather_kernel(n, off, es)(a_p, b_p, row, col)
        msg = _msg_tc(edge_features, ar, bc, W_edge, r2(b_edge),
                      r2(g_edge), r2(be_edge), w2, r2(g_msg), r2(be_msg),
                      off)
        parts.append(_sc_scatter_kernel(n, off, es)(msg, row, col, zeros))
    out = _finish(x, parts[0][0, :n], parts[0][1, :n],
                  parts[1][0, :n], parts[1][1, :n],
                  batch.astype(jnp.int32).reshape(n, 1),
                  W_upd[:H], W_upd[H:], r2(b_upd), r2(g_upd), r2(be_upd),
                  W_o1, r2(b_o1), r2(g_o), r2(be_o), W_o2, r2(b_o2))
    return out
